# baseline (device time: 64151 ns/iter reference)
import jax
import jax.numpy as jnp
from jax import lax
from jax.experimental import pallas as pl
from jax.experimental.pallas import tpu as pltpu

N_DEV = 16
SQ = 1024
D = 1024
HQ_PER = 8
DH = 128
HD = HQ_PER * DH
SCALE = 0.08838834764831843
WINDOW = 128
QB = 256
KW = 2 * QB

PLANE_G = [1, 2, 3]
Z_G = [4, 8, 12]
RS_GROUPS = [[PLANE_G, Z_G], [Z_G, PLANE_G]]
AG_GROUPS = [[Z_G, PLANE_G], [PLANE_G, Z_G]]
RS_QS = [128, 32]
AG_QS = [32, 128]
PART_BASE = [0, 512]
SOFF = [0, 384]
RS_OFF = [[0, 768], [384, 864]]
AG_OFF = [[0, 192], [96, 576]]
WIRE_DTYPE = jnp.bfloat16
BF = jnp.bfloat16
F32 = jnp.float32


def _digit(e, group):
    if group is PLANE_G:
        return jnp.bitwise_and(e, 3)
    return jnp.bitwise_and(e // 4, 3)


def _fused(x, Wq_my, KT, V, Wo_my):

    def body(x_ref, wq_ref, kt_ref, v_ref, wo_ref, out_ref,
             p_ref, q_ref, ctx_ref, sbuf, rsbuf, agbuf,
             rs_send, rs_recv, ag_send, ag_recv):
        my = lax.axis_index("i")

        barrier = pltpu.get_barrier_semaphore()
        for m in PLANE_G + Z_G:
            pl.semaphore_signal(
                barrier, inc=1,
                device_id=(jnp.bitwise_xor(my, m),),
                device_id_type=pl.DeviceIdType.MESH,
            )
        pl.semaphore_wait(barrier, 6)

        q_ref[...] = (
            jnp.dot(x_ref[...], wq_ref[...], preferred_element_type=F32)
            * SCALE
        ).astype(BF)

        def attn_block(i):
            k_lo = min(max(0, QB * i - WINDOW), SQ - KW)
            qi = QB * i + lax.broadcasted_iota(jnp.int32, (QB, KW), 0)
            ki = k_lo + lax.broadcasted_iota(jnp.int32, (QB, KW), 1)
            neg = jnp.where(
                jnp.abs(qi - ki) <= WINDOW, jnp.float32(0.0),
                jnp.float32(-1e9),
            )
            for h in range(HQ_PER):
                qblk = q_ref[QB * i : QB * (i + 1), DH * h : DH * (h + 1)]
                s = lax.dot_general(
                    qblk, kt_ref[h, :, k_lo : k_lo + KW],
                    (((1,), (0,)), ((), ())),
                    preferred_element_type=F32,
                ) + neg
                w = jnp.exp(s)
                r = 1.0 / jnp.sum(w, axis=1, keepdims=True)
                ctx_ref[QB * i : QB * (i + 1), DH * h : DH * (h + 1)] = (
                    lax.dot_general(
                        w.astype(BF), v_ref[h, k_lo : k_lo + KW, :],
                        (((1,), (0,)), ((), ())),
                        preferred_element_type=F32,
                    ) * r
                ).astype(BF)

        lo = [jnp.int32(PART_BASE[0]), jnp.int32(PART_BASE[1])]
        qs = RS_QS[0]
        rdmas0 = [[], []]
        keeps0 = [None, None]

        def rs0_launch(t):
            grp = RS_GROUPS[t][0]
            keeps0[t] = lo[t] + _digit(my, grp) * qs
            for idx, m in enumerate(grp):
                peer = jnp.bitwise_xor(my, m)
                send_lo = lo[t] + _digit(peer, grp) * qs
                so = SOFF[t] + idx * qs
                sbuf[pl.ds(so, qs), :] = jnp.dot(
                    ctx_ref[pl.ds(send_lo, qs), :], wo_ref[...],
                    preferred_element_type=F32,
                ).astype(WIRE_DTYPE)
                rdma = pltpu.make_async_remote_copy(
                    src_ref=sbuf.at[pl.ds(so, qs), :],
                    dst_ref=rsbuf.at[pl.ds(RS_OFF[t][0] + idx * qs, qs), :],
                    send_sem=rs_send.at[3 * t + idx],
                    recv_sem=rs_recv.at[3 * t + idx],
                    device_id=(peer,),
                    device_id_type=pl.DeviceIdType.MESH,
                )
                rdma.start()
                rdmas0[t].append(rdma)

        attn_block(0)
        attn_block(1)
        rs0_launch(0)
        attn_block(2)
        attn_block(3)
        rs0_launch(1)
        for t in range(2):
            p_ref[pl.ds(keeps0[t], qs), :] = jnp.dot(
                ctx_ref[pl.ds(keeps0[t], qs), :], wo_ref[...],
                preferred_element_type=F32,
            )
        for t in range(2):
            for r in rdmas0[t]:
                r.wait()
            base = RS_OFF[t][0]
            out_ref[pl.ds(keeps0[t], qs), :] = (
                p_ref[pl.ds(keeps0[t], qs), :]
                + rsbuf[base : base + qs, :].astype(F32)
                + rsbuf[base + qs : base + 2 * qs, :].astype(F32)
                + rsbuf[base + 2 * qs : base + 3 * qs, :].astype(F32)
            )
            lo[t] = keeps0[t]

        qs = RS_QS[1]
        rdmas1 = [[], []]
        keeps1 = []
        for t in range(2):
            grp = RS_GROUPS[t][1]
            keeps1.append(lo[t] + _digit(my, grp) * qs)
            for idx, m in enumerate(grp):
                peer = jnp.bitwise_xor(my, m)
                send_lo = lo[t] + _digit(peer, grp) * qs
                so = SOFF[t] + idx * qs
                sbuf[pl.ds(so, qs), :] = out_ref[pl.ds(send_lo, qs), :].astype(
                    WIRE_DTYPE
                )
                rdma = pltpu.make_async_remote_copy(
                    src_ref=sbuf.at[pl.ds(so, qs), :],
                    dst_ref=rsbuf.at[pl.ds(RS_OFF[t][1] + idx * qs, qs), :],
                    send_sem=rs_send.at[6 + 3 * t + idx],
                    recv_sem=rs_recv.at[6 + 3 * t + idx],
                    device_id=(peer,),
                    device_id_type=pl.DeviceIdType.MESH,
                )
                rdma.start()
                rdmas1[t].append(rdma)
        for t in range(2):
            for r in rdmas1[t]:
                r.wait()
            base = RS_OFF[t][1]
            out_ref[pl.ds(keeps1[t], qs), :] = (
                out_ref[pl.ds(keeps1[t], qs), :]
                + rsbuf[base : base + qs, :].astype(F32)
                + rsbuf[base + qs : base + 2 * qs, :].astype(F32)
                + rsbuf[base + 2 * qs : base + 3 * qs, :].astype(F32)
            )
            lo[t] = keeps1[t]

        for j in range(2):
            n = AG_QS[j]
            rdmas = [[], []]
            bases = []
            grps = []
            for t in range(2):
                grp = AG_GROUPS[t][j]
                grps.append(grp)
                bases.append(lo[t] - _digit(my, grp) * n)
                sbuf[pl.ds(SOFF[t], n), :] = out_ref[pl.ds(lo[t], n), :].astype(
                    WIRE_DTYPE
                )
                for idx, m in enumerate(grp):
                    rdma = pltpu.make_async_remote_copy(
                        src_ref=sbuf.at[pl.ds(SOFF[t], n), :],
                        dst_ref=agbuf.at[pl.ds(AG_OFF[t][j] + idx * n, n), :],
                        send_sem=ag_send.at[6 * j + 3 * t + idx],
                        recv_sem=ag_recv.at[6 * j + 3 * t + idx],
                        device_id=(jnp.bitwise_xor(my, m),),
                        device_id_type=pl.DeviceIdType.MESH,
                    )
                    rdma.start()
                    rdmas[t].append(rdma)
            for t in range(2):
                for idx, m in enumerate(grps[t]):
                    rdmas[t][idx].wait()
                    peer = jnp.bitwise_xor(my, m)
                    dst_lo = bases[t] + _digit(peer, grps[t]) * n
                    out_ref[pl.ds(dst_lo, n), :] = agbuf[
                        pl.ds(AG_OFF[t][j] + idx * n, n), :
                    ].astype(F32)
                lo[t] = bases[t]

    return pl.pallas_call(
        body,
        out_shape=jax.ShapeDtypeStruct((SQ, D), F32),
        in_specs=[pl.BlockSpec(memory_space=pltpu.VMEM)] * 5,
        out_specs=pl.BlockSpec(memory_space=pltpu.VMEM),
        scratch_shapes=[
            pltpu.VMEM((SQ, D), F32),
            pltpu.VMEM((SQ, HD), BF),
            pltpu.VMEM((SQ, HD), BF),
            pltpu.VMEM((768, D), WIRE_DTYPE),
            pltpu.VMEM((960, D), WIRE_DTYPE),
            pltpu.VMEM((960, D), WIRE_DTYPE),
            pltpu.SemaphoreType.DMA((12,)),
            pltpu.SemaphoreType.DMA((12,)),
            pltpu.SemaphoreType.DMA((12,)),
            pltpu.SemaphoreType.DMA((12,)),
        ],
        compiler_params=pltpu.CompilerParams(collective_id=0),
    )(x, Wq_my, KT, V, Wo_my)


def kernel(x, Wq, K_ext, V_ext, Wo):
    pos = lax.axis_index("i")

    Wq_my = lax.dynamic_slice(Wq, (0, pos * HD), (D, HD))
    Wo_my = lax.dynamic_slice(Wo, (pos * HD, 0), (HD, D))

    xb = x[0].astype(BF)
    wqb = Wq_my.astype(BF)
    wob = Wo_my.astype(BF)
    KT = jnp.transpose(K_ext[0].astype(BF), (1, 2, 0))
    Vh = jnp.transpose(V_ext[0].astype(BF), (1, 0, 2))

    out = _fused(xb, wqb, KT, Vh, wob)
    return out[None]


# device time: 64135 ns/iter; 1.0002x vs baseline; 1.0002x over previous
import jax
import jax.numpy as jnp
from jax import lax
from jax.experimental import pallas as pl
from jax.experimental.pallas import tpu as pltpu

N_DEV = 16
SQ = 1024
D = 1024
HQ_PER = 8
DH = 128
HD = HQ_PER * DH
SCALE = 0.08838834764831843
WINDOW = 128
QB = 256
KW = 2 * QB

PLANE_G = [1, 2, 3]
Z_G = [4, 8, 12]
RS_GROUPS = [[PLANE_G, Z_G], [Z_G, PLANE_G]]
AG_GROUPS = [[Z_G, PLANE_G], [PLANE_G, Z_G]]
RS_QS = [128, 32]
AG_QS = [32, 128]
PART_BASE = [0, 512]
SOFF = [0, 384]
RS_OFF = [[0, 768], [384, 864]]
AG_OFF = [[0, 192], [96, 576]]
WIRE_DTYPE = jnp.bfloat16
BF = jnp.bfloat16
F32 = jnp.float32


def _digit(e, group):
    if group is PLANE_G:
        return jnp.bitwise_and(e, 3)
    return jnp.bitwise_and(e // 4, 3)


def _fused(x, Wq_my, KT, V, Wo_my):

    def body(x_ref, wq_ref, kt_ref, v_ref, wo_ref, out_ref,
             p_ref, q_ref, ctx_ref, sbuf, rsbuf, agbuf,
             rs_send, rs_recv, ag_send, ag_recv):
        my = lax.axis_index("i")

        barrier = pltpu.get_barrier_semaphore()
        for m in PLANE_G + Z_G:
            pl.semaphore_signal(
                barrier, inc=1,
                device_id=(jnp.bitwise_xor(my, m),),
                device_id_type=pl.DeviceIdType.MESH,
            )
        pl.semaphore_wait(barrier, 6)

        q_ref[...] = (
            jnp.dot(x_ref[...], wq_ref[...], preferred_element_type=F32)
            * (SCALE * 1.4426950408889634)
        ).astype(BF)

        def attn_block(i):
            k_lo = min(max(0, QB * i - WINDOW), SQ - KW)
            qi = QB * i + lax.broadcasted_iota(jnp.int32, (QB, KW), 0)
            ki = k_lo + lax.broadcasted_iota(jnp.int32, (QB, KW), 1)
            neg = jnp.where(
                jnp.abs(qi - ki) <= WINDOW, jnp.float32(0.0),
                jnp.float32(-1e9),
            )
            for h in range(HQ_PER):
                qblk = q_ref[QB * i : QB * (i + 1), DH * h : DH * (h + 1)]
                s = lax.dot_general(
                    qblk, kt_ref[h, :, k_lo : k_lo + KW],
                    (((1,), (0,)), ((), ())),
                    preferred_element_type=F32,
                ) + neg
                w = jnp.exp2(s)
                r = 1.0 / jnp.sum(w, axis=1, keepdims=True)
                ctx_ref[QB * i : QB * (i + 1), DH * h : DH * (h + 1)] = (
                    lax.dot_general(
                        w.astype(BF), v_ref[h, k_lo : k_lo + KW, :],
                        (((1,), (0,)), ((), ())),
                        preferred_element_type=F32,
                    ) * r
                ).astype(BF)

        lo = [jnp.int32(PART_BASE[0]), jnp.int32(PART_BASE[1])]
        qs = RS_QS[0]
        rdmas0 = [[], []]
        keeps0 = [None, None]

        def rs0_launch(t):
            grp = RS_GROUPS[t][0]
            keeps0[t] = lo[t] + _digit(my, grp) * qs
            for idx, m in enumerate(grp):
                peer = jnp.bitwise_xor(my, m)
                send_lo = lo[t] + _digit(peer, grp) * qs
                so = SOFF[t] + idx * qs
                sbuf[pl.ds(so, qs), :] = jnp.dot(
                    ctx_ref[pl.ds(send_lo, qs), :], wo_ref[...],
                    preferred_element_type=F32,
                ).astype(WIRE_DTYPE)
                rdma = pltpu.make_async_remote_copy(
                    src_ref=sbuf.at[pl.ds(so, qs), :],
                    dst_ref=rsbuf.at[pl.ds(RS_OFF[t][0] + idx * qs, qs), :],
                    send_sem=rs_send.at[3 * t + idx],
                    recv_sem=rs_recv.at[3 * t + idx],
                    device_id=(peer,),
                    device_id_type=pl.DeviceIdType.MESH,
                )
                rdma.start()
                rdmas0[t].append(rdma)

        attn_block(0)
        attn_block(1)
        rs0_launch(0)
        attn_block(2)
        attn_block(3)
        rs0_launch(1)
        for t in range(2):
            p_ref[pl.ds(keeps0[t], qs), :] = jnp.dot(
                ctx_ref[pl.ds(keeps0[t], qs), :], wo_ref[...],
                preferred_element_type=F32,
            )
        for t in range(2):
            for r in rdmas0[t]:
                r.wait()
            base = RS_OFF[t][0]
            out_ref[pl.ds(keeps0[t], qs), :] = (
                p_ref[pl.ds(keeps0[t], qs), :]
                + rsbuf[base : base + qs, :].astype(F32)
                + rsbuf[base + qs : base + 2 * qs, :].astype(F32)
                + rsbuf[base + 2 * qs : base + 3 * qs, :].astype(F32)
            )
            lo[t] = keeps0[t]

        qs = RS_QS[1]
        rdmas1 = [[], []]
        keeps1 = []
        for t in range(2):
            grp = RS_GROUPS[t][1]
            keeps1.append(lo[t] + _digit(my, grp) * qs)
            for idx, m in enumerate(grp):
                peer = jnp.bitwise_xor(my, m)
                send_lo = lo[t] + _digit(peer, grp) * qs
                so = SOFF[t] + idx * qs
                sbuf[pl.ds(so, qs), :] = out_ref[pl.ds(send_lo, qs), :].astype(
                    WIRE_DTYPE
                )
                rdma = pltpu.make_async_remote_copy(
                    src_ref=sbuf.at[pl.ds(so, qs), :],
                    dst_ref=rsbuf.at[pl.ds(RS_OFF[t][1] + idx * qs, qs), :],
                    send_sem=rs_send.at[6 + 3 * t + idx],
                    recv_sem=rs_recv.at[6 + 3 * t + idx],
                    device_id=(peer,),
                    device_id_type=pl.DeviceIdType.MESH,
                )
                rdma.start()
                rdmas1[t].append(rdma)
        for t in range(2):
            for r in rdmas1[t]:
                r.wait()
            base = RS_OFF[t][1]
            out_ref[pl.ds(keeps1[t], qs), :] = (
                out_ref[pl.ds(keeps1[t], qs), :]
                + rsbuf[base : base + qs, :].astype(F32)
                + rsbuf[base + qs : base + 2 * qs, :].astype(F32)
                + rsbuf[base + 2 * qs : base + 3 * qs, :].astype(F32)
            )
            lo[t] = keeps1[t]

        for j in range(2):
            n = AG_QS[j]
            rdmas = [[], []]
            bases = []
            grps = []
            for t in range(2):
                grp = AG_GROUPS[t][j]
                grps.append(grp)
                bases.append(lo[t] - _digit(my, grp) * n)
                sbuf[pl.ds(SOFF[t], n), :] = out_ref[pl.ds(lo[t], n), :].astype(
                    WIRE_DTYPE
                )
                for idx, m in enumerate(grp):
                    rdma = pltpu.make_async_remote_copy(
                        src_ref=sbuf.at[pl.ds(SOFF[t], n), :],
                        dst_ref=agbuf.at[pl.ds(AG_OFF[t][j] + idx * n, n), :],
                        send_sem=ag_send.at[6 * j + 3 * t + idx],
                        recv_sem=ag_recv.at[6 * j + 3 * t + idx],
                        device_id=(jnp.bitwise_xor(my, m),),
                        device_id_type=pl.DeviceIdType.MESH,
                    )
                    rdma.start()
                    rdmas[t].append(rdma)
            for t in range(2):
                for idx, m in enumerate(grps[t]):
                    rdmas[t][idx].wait()
                    peer = jnp.bitwise_xor(my, m)
                    dst_lo = bases[t] + _digit(peer, grps[t]) * n
                    out_ref[pl.ds(dst_lo, n), :] = agbuf[
                        pl.ds(AG_OFF[t][j] + idx * n, n), :
                    ].astype(F32)
                lo[t] = bases[t]

    return pl.pallas_call(
        body,
        out_shape=jax.ShapeDtypeStruct((SQ, D), F32),
        in_specs=[pl.BlockSpec(memory_space=pltpu.VMEM)] * 5,
        out_specs=pl.BlockSpec(memory_space=pltpu.VMEM),
        scratch_shapes=[
            pltpu.VMEM((SQ, D), F32),
            pltpu.VMEM((SQ, HD), BF),
            pltpu.VMEM((SQ, HD), BF),
            pltpu.VMEM((768, D), WIRE_DTYPE),
            pltpu.VMEM((960, D), WIRE_DTYPE),
            pltpu.VMEM((960, D), WIRE_DTYPE),
            pltpu.SemaphoreType.DMA((12,)),
            pltpu.SemaphoreType.DMA((12,)),
            pltpu.SemaphoreType.DMA((12,)),
            pltpu.SemaphoreType.DMA((12,)),
        ],
        compiler_params=pltpu.CompilerParams(collective_id=0),
    )(x, Wq_my, KT, V, Wo_my)


def kernel(x, Wq, K_ext, V_ext, Wo):
    pos = lax.axis_index("i")

    Wq_my = lax.dynamic_slice(Wq, (0, pos * HD), (D, HD))
    Wo_my = lax.dynamic_slice(Wo, (pos * HD, 0), (HD, D))

    xb = x[0].astype(BF)
    wqb = Wq_my.astype(BF)
    wob = Wo_my.astype(BF)
    KT = jnp.transpose(K_ext[0].astype(BF), (1, 2, 0))
    Vh = jnp.transpose(V_ext[0].astype(BF), (1, 0, 2))

    out = _fused(xb, wqb, KT, Vh, wob)
    return out[None]
